# baseline (speedup 1.0000x reference)
import functools

import jax
import jax.numpy as jnp
from jax import lax
from jax.experimental import pallas as pl
from jax.experimental.pallas import tpu as pltpu
from jax.experimental.pallas import tpu_sc as plsc

_B, _N = 64, 32768
_K1 = int(0.01 * (_N - 1)) + 1
_NVEC = _N // 16
_CAP = 32000
_CAPH = 16000
_MASK31 = 0x7FFFFFFF
_TAU = 0x40000000


def _sc_select_body(logits_hbm, out_hbm, row0_v, row1_v, cand0_v, cand1_v,
                    ebuf_v, sem):
    wid = lax.axis_index("s") * 2 + lax.axis_index("c")
    r0 = wid * 2
    keys_hbm = logits_hbm.bitcast(jnp.int32)
    pltpu.sync_copy(keys_hbm.at[r0], row0_v)
    h1 = pltpu.async_copy(keys_hbm.at[r0 + 1], row1_v, sem)

    def screen(row_v, cand_v):
        @plsc.parallel_loop(0, _NVEC // 2, unroll=4,
                            carry=(jnp.int32(0), jnp.int32(0)))
        def offs(i, c):
            oa, ob = c
            ba = row_v[pl.ds(i * 16, 16)]
            ka = ba ^ ((ba >> 31) & _MASK31)
            ma = ka >= _TAU
            plsc.store_compressed(cand_v.at[pl.ds(oa, 16)], ka,
                                  mask=ma & (oa < _CAPH))
            bb = row_v[pl.ds(_N // 2 + i * 16, 16)]
            kb = bb ^ ((bb >> 31) & _MASK31)
            mb = kb >= _TAU
            plsc.store_compressed(cand_v.at[pl.ds(_CAPH + 16 + ob, 16)], kb,
                                  mask=mb & (ob < _CAPH))
            return (oa + plsc.all_reduce_population_count(ma)[0],
                    ob + plsc.all_reduce_population_count(mb)[0])

        return offs

    offs0 = screen(row0_v, cand0_v)
    h1.wait()
    offs1 = screen(row1_v, cand1_v)

    pad = jnp.full((16,), -2147483648, jnp.int32)

    for r_i, (row_v, cand_v, offs) in enumerate(
            [(row0_v, cand0_v, offs0), (row1_v, cand1_v, offs1)]):
        oa, ob = offs
        off = oa + ob

        def fast_path(_, cand_v=cand_v, oa=oa, ob=ob):
            cand_v[pl.ds(oa, 16)] = pad
            cand_v[pl.ds(_CAPH + 16 + ob, 16)] = pad
            nva = (oa + 15) // 16
            nvb = (ob + 15) // 16

            def outer(j, res):
                cnd = res + (jnp.int32(1) << (jnp.int32(29) - j))

                @plsc.parallel_loop(0, nva, unroll=4,
                                    carry=jnp.zeros((16,), jnp.int32))
                def acca(i, a, cnd=cnd, cand_v=cand_v):
                    key = cand_v[pl.ds(i * 16, 16)]
                    return a + jnp.where(key >= cnd, 1, 0).astype(jnp.int32)

                @plsc.parallel_loop(0, nvb, unroll=4, carry=acca)
                def accb(i, a, cnd=cnd, cand_v=cand_v):
                    key = cand_v[pl.ds(_CAPH + 16 + i * 16, 16)]
                    return a + jnp.where(key >= cnd, 1, 0).astype(jnp.int32)

                return jnp.where(jnp.sum(accb) >= _K1, cnd, res)

            return lax.fori_loop(0, 30, outer, jnp.int32(_TAU))

        def slow_path(_, row_v=row_v):
            def outer(j, res):
                cnd = res + (jnp.int32(1) << (jnp.int32(30) - j))

                @plsc.parallel_loop(0, _NVEC, unroll=4,
                                    carry=jnp.zeros((16,), jnp.int32))
                def acc(i, a, cnd=cnd, row_v=row_v):
                    b = row_v[pl.ds(i * 16, 16)]
                    key = b ^ ((b >> 31) & _MASK31)
                    return a + jnp.where(key >= cnd, 1, 0).astype(jnp.int32)

                return jnp.where(jnp.sum(acc) >= _K1, cnd, res)

            return lax.fori_loop(0, 31, outer, jnp.int32(-2147483648))

        ok = (off >= _K1) & (oa <= _CAPH) & (ob <= _CAPH)
        tcode = lax.cond(ok, fast_path, slow_path, jnp.int32(0))
        ebuf_v[pl.ds(0, 16)] = jnp.full((16,), tcode)
        pltpu.sync_copy(ebuf_v, out_hbm.at[r0 + r_i])


def _tc_stats_body(logits_ref, tgt_ref, m_ref, pos_ref):
    x = logits_ref[...]
    m_ref[...] = jnp.max(x, axis=1, keepdims=True)
    cols = lax.broadcasted_iota(jnp.int32, (_B, _N), 1)
    pos_ref[...] = jnp.sum(jnp.where(cols == tgt_ref[...], x, 0.0), axis=1,
                           keepdims=True)


def _tc_finish_body(logits_ref, tcode_ref, m_ref, pos_ref, out_ref):
    x = logits_ref[...]
    b = lax.bitcast_convert_type(x, jnp.int32)
    key = b ^ ((b >> 31) & _MASK31)
    tcode = tcode_ref[:, 0:1]
    m = m_ref[...]
    pos = pos_ref[...]

    gt = key > tcode
    c = jnp.sum(gt.astype(jnp.int32), axis=1, keepdims=True)
    tsum = jnp.sum(jnp.where(gt, jnp.exp(10.0 * (x - m)), 0.0), axis=1,
                   keepdims=True)

    tb = tcode ^ ((tcode >> 31) & _MASK31)
    t = lax.bitcast_convert_type(tb, jnp.float32)
    et = jnp.exp(10.0 * (t - m))
    ep = jnp.exp(10.0 * (pos - m))
    s = tsum + (_K1 - c).astype(jnp.float32) * et + jnp.where(
        pos < t, ep - et, 0.0)
    loss = jnp.log(s) + 10.0 * m - 10.0 * pos
    out_ref[0, 0] = jnp.sum(loss) / _B


@jax.jit
def kernel(logits, targets):
    mesh = plsc.VectorSubcoreMesh(core_axis_name="c", subcore_axis_name="s")
    sc = functools.partial(
        pl.kernel,
        mesh=mesh,
        compiler_params=pltpu.CompilerParams(needs_layout_passes=False),
        out_type=jax.ShapeDtypeStruct((_B, 16), jnp.int32),
        scratch_types=[
            pltpu.VMEM((_N,), jnp.int32),
            pltpu.VMEM((_N,), jnp.int32),
            pltpu.VMEM((2 * (_CAPH + 16),), jnp.int32),
            pltpu.VMEM((2 * (_CAPH + 16),), jnp.int32),
            pltpu.VMEM((16,), jnp.int32),
            pltpu.SemaphoreType.DMA,
        ],
    )(_sc_select_body)
    tcodes = sc(logits)

    m, pos = pl.pallas_call(
        _tc_stats_body,
        out_shape=[
            jax.ShapeDtypeStruct((_B, 1), jnp.float32),
            jax.ShapeDtypeStruct((_B, 1), jnp.float32),
        ],
        in_specs=[
            pl.BlockSpec(memory_space=pltpu.VMEM),
            pl.BlockSpec(memory_space=pltpu.VMEM),
        ],
        out_specs=[
            pl.BlockSpec(memory_space=pltpu.VMEM),
            pl.BlockSpec(memory_space=pltpu.VMEM),
        ],
    )(logits, targets.reshape(_B, 1).astype(jnp.int32))

    out = pl.pallas_call(
        _tc_finish_body,
        out_shape=jax.ShapeDtypeStruct((1, 1), jnp.float32),
        in_specs=[
            pl.BlockSpec(memory_space=pltpu.VMEM),
            pl.BlockSpec(memory_space=pltpu.VMEM),
            pl.BlockSpec(memory_space=pltpu.VMEM),
            pl.BlockSpec(memory_space=pltpu.VMEM),
        ],
        out_specs=pl.BlockSpec(memory_space=pltpu.SMEM),
    )(logits, tcodes, m, pos)
    return out[0, 0]


# speedup vs baseline: 1.0209x; 1.0209x over previous
import functools

import jax
import jax.numpy as jnp
from jax import lax
from jax.experimental import pallas as pl
from jax.experimental.pallas import tpu as pltpu
from jax.experimental.pallas import tpu_sc as plsc

_B, _N = 64, 32768
_K1 = int(0.01 * (_N - 1)) + 1
_NVEC = _N // 16
_CAP = 32000
_MASK31 = 0x7FFFFFFF
_TAU = 0x40000000


def _sc_select_body(logits_hbm, out_hbm, row0_v, row1_v, cand0_v, cand1_v,
                    ebuf_v, sem):
    wid = lax.axis_index("s") * 2 + lax.axis_index("c")
    r0 = wid * 2
    keys_hbm = logits_hbm.bitcast(jnp.int32)
    pltpu.sync_copy(keys_hbm.at[r0], row0_v)
    h1 = pltpu.async_copy(keys_hbm.at[r0 + 1], row1_v, sem)

    def screen(row_v, cand_v):
        @plsc.parallel_loop(0, _NVEC, unroll=8, carry=jnp.int32(0))
        def off(i, o):
            b = row_v[pl.ds(i * 16, 16)]
            k = b ^ ((b >> 31) & _MASK31)
            m = k >= _TAU
            plsc.store_compressed(cand_v.at[pl.ds(o, 16)], k,
                                  mask=m & (o < _CAP))
            return o + plsc.all_reduce_population_count(m)[0]

        return off

    off0 = screen(row0_v, cand0_v)
    h1.wait()
    off1 = screen(row1_v, cand1_v)

    pad = jnp.full((16,), -2147483648, jnp.int32)

    for r_i, (row_v, cand_v, off) in enumerate(
            [(row0_v, cand0_v, off0), (row1_v, cand1_v, off1)]):

        def fast_path(_, cand_v=cand_v, off=off):
            cand_v[pl.ds(off, 16)] = pad
            nv = (off + 15) // 16

            def outer(j, res):
                cnd = res + (jnp.int32(1) << (jnp.int32(29) - j))

                @plsc.parallel_loop(0, nv, unroll=4,
                                    carry=jnp.zeros((16,), jnp.int32))
                def acc(i, a, cnd=cnd, cand_v=cand_v):
                    key = cand_v[pl.ds(i * 16, 16)]
                    return a + jnp.where(key >= cnd, 1, 0).astype(jnp.int32)

                return jnp.where(jnp.sum(acc) >= _K1, cnd, res)

            return lax.fori_loop(0, 30, outer, jnp.int32(_TAU))

        def slow_path(_, row_v=row_v):
            def outer(j, res):
                cnd = res + (jnp.int32(1) << (jnp.int32(30) - j))

                @plsc.parallel_loop(0, _NVEC, unroll=4,
                                    carry=jnp.zeros((16,), jnp.int32))
                def acc(i, a, cnd=cnd, row_v=row_v):
                    b = row_v[pl.ds(i * 16, 16)]
                    key = b ^ ((b >> 31) & _MASK31)
                    return a + jnp.where(key >= cnd, 1, 0).astype(jnp.int32)

                return jnp.where(jnp.sum(acc) >= _K1, cnd, res)

            return lax.fori_loop(0, 31, outer, jnp.int32(-2147483648))

        ok = (off >= _K1) & (off <= _CAP)
        tcode = lax.cond(ok, fast_path, slow_path, jnp.int32(0))
        ebuf_v[pl.ds(0, 16)] = jnp.full((16,), tcode)
        pltpu.sync_copy(ebuf_v, out_hbm.at[r0 + r_i])


def _tc_stats_body(logits_ref, tgt_ref, m_ref, pos_ref):
    x = logits_ref[...]
    m_ref[...] = jnp.max(x, axis=1, keepdims=True)
    cols = lax.broadcasted_iota(jnp.int32, (_B, _N), 1)
    pos_ref[...] = jnp.sum(jnp.where(cols == tgt_ref[...], x, 0.0), axis=1,
                           keepdims=True)


def _tc_finish_body(logits_ref, tcode_ref, m_ref, pos_ref, out_ref):
    x = logits_ref[...]
    b = lax.bitcast_convert_type(x, jnp.int32)
    key = b ^ ((b >> 31) & _MASK31)
    tcode = tcode_ref[:, 0:1]
    m = m_ref[...]
    pos = pos_ref[...]

    gt = key > tcode
    c = jnp.sum(gt.astype(jnp.int32), axis=1, keepdims=True)
    tsum = jnp.sum(jnp.where(gt, jnp.exp(10.0 * (x - m)), 0.0), axis=1,
                   keepdims=True)

    tb = tcode ^ ((tcode >> 31) & _MASK31)
    t = lax.bitcast_convert_type(tb, jnp.float32)
    et = jnp.exp(10.0 * (t - m))
    ep = jnp.exp(10.0 * (pos - m))
    s = tsum + (_K1 - c).astype(jnp.float32) * et + jnp.where(
        pos < t, ep - et, 0.0)
    loss = jnp.log(s) + 10.0 * m - 10.0 * pos
    out_ref[0, 0] = jnp.sum(loss) / _B


@jax.jit
def kernel(logits, targets):
    mesh = plsc.VectorSubcoreMesh(core_axis_name="c", subcore_axis_name="s")
    sc = functools.partial(
        pl.kernel,
        mesh=mesh,
        compiler_params=pltpu.CompilerParams(needs_layout_passes=False),
        out_type=jax.ShapeDtypeStruct((_B, 16), jnp.int32),
        scratch_types=[
            pltpu.VMEM((_N,), jnp.int32),
            pltpu.VMEM((_N,), jnp.int32),
            pltpu.VMEM((_CAP + 16,), jnp.int32),
            pltpu.VMEM((_CAP + 16,), jnp.int32),
            pltpu.VMEM((16,), jnp.int32),
            pltpu.SemaphoreType.DMA,
        ],
    )(_sc_select_body)
    tcodes = sc(logits)

    m, pos = pl.pallas_call(
        _tc_stats_body,
        out_shape=[
            jax.ShapeDtypeStruct((_B, 1), jnp.float32),
            jax.ShapeDtypeStruct((_B, 1), jnp.float32),
        ],
        in_specs=[
            pl.BlockSpec(memory_space=pltpu.VMEM),
            pl.BlockSpec(memory_space=pltpu.VMEM),
        ],
        out_specs=[
            pl.BlockSpec(memory_space=pltpu.VMEM),
            pl.BlockSpec(memory_space=pltpu.VMEM),
        ],
    )(logits, targets.reshape(_B, 1).astype(jnp.int32))

    out = pl.pallas_call(
        _tc_finish_body,
        out_shape=jax.ShapeDtypeStruct((1, 1), jnp.float32),
        in_specs=[
            pl.BlockSpec(memory_space=pltpu.VMEM),
            pl.BlockSpec(memory_space=pltpu.VMEM),
            pl.BlockSpec(memory_space=pltpu.VMEM),
            pl.BlockSpec(memory_space=pltpu.VMEM),
        ],
        out_specs=pl.BlockSpec(memory_space=pltpu.SMEM),
    )(logits, tcodes, m, pos)
    return out[0, 0]
